# baseline (speedup 1.0000x reference)
import functools

import jax
import jax.numpy as jnp
from jax import lax
from jax.experimental import pallas as pl
from jax.experimental.pallas import tpu as pltpu
from jax.experimental.pallas import tpu_sc as plsc

_NC = 2
_NS = 16
_LANES = 16


def _proj_body(tableT_ref, w_ref, b_ref, v_ref):
    t = tableT_ref[...]
    w = w_ref[...]
    r = lax.dot_general(
        w, t, (((1,), (0,)), ((), ())), preferred_element_type=jnp.float32
    )
    v_ref[...] = jnp.reshape(r, (r.shape[1],)) + b_ref[0]


def _project(tableT, w_row, b, col_block):
    embed, vocab = tableT.shape
    grid = pl.cdiv(vocab, col_block)
    return pl.pallas_call(
        _proj_body,
        grid=(grid,),
        in_specs=[
            pl.BlockSpec((embed, col_block), lambda i: (0, i)),
            pl.BlockSpec((1, embed), lambda i: (0, 0)),
            pl.BlockSpec(memory_space=pltpu.SMEM),
        ],
        out_specs=pl.BlockSpec((col_block,), lambda i: (i,)),
        out_shape=jax.ShapeDtypeStruct((vocab,), jnp.float32),
    )(tableT, w_row, b)


def _make_pool(vocab, batch, seq, chunk_cols, unroll):
    nw = _NC * _NS
    cols_per_w = batch // nw
    n_chunks = cols_per_w // chunk_cols
    groups = chunk_cols // _LANES
    halves = (40, 40, 40, 40, 40)
    hoffs = (0, 40, 80, 120, 160)
    nh = len(halves)
    nbuf = 4
    n_pieces = nh * n_chunks
    mesh = plsc.VectorSubcoreMesh(core_axis_name="c", subcore_axis_name="s")

    @functools.partial(
        pl.kernel,
        out_type=jax.ShapeDtypeStruct((batch,), jnp.float32),
        mesh=mesh,
        compiler_params=pltpu.CompilerParams(needs_layout_passes=False),
        scratch_types=[
            pltpu.VMEM((vocab,), jnp.float32),
            pltpu.VMEM_SHARED((vocab,), jnp.float32),
            pltpu.VMEM((4, max(halves), chunk_cols), jnp.int32),
            pltpu.VMEM((cols_per_w,), jnp.float32),
            pltpu.SemaphoreType.DMA,
            pltpu.SemaphoreType.DMA,
            pltpu.SemaphoreType.DMA,
            pltpu.SemaphoreType.DMA,
            pltpu.SemaphoreType.DMA,
        ],
    )
    def pool(
        v_hbm, xT_hbm, out_hbm, v_vmem, v_shared, idxb, out_vmem,
        vsem, s0, s1, s2, s3,
    ):
        sid = lax.axis_index("s")
        wid = sid * _NC + lax.axis_index("c")
        col0 = wid * cols_per_w
        sems = (s0, s1, s2, s3)

        def start(p):
            c, h = p // nh, p % nh
            return pltpu.async_copy(
                xT_hbm.at[
                    pl.ds(hoffs[h], halves[h]),
                    pl.ds(col0 + c * chunk_cols, chunk_cols),
                ],
                idxb.at[p % nbuf, pl.ds(0, halves[h])],
                sems[p % nbuf],
            )

        inv = jnp.float32(1.0 / seq)
        copies = [None] * nbuf
        for p0 in range(nbuf):
            copies[p0] = start(p0)

        @pl.when(sid == 0)
        def _stage_v():
            pltpu.sync_copy(v_hbm, v_shared)

        plsc.subcore_barrier()
        vcopy = pltpu.async_copy(v_shared, v_vmem, vsem)
        for p in range(n_pieces):
            c, h = p // nh, p % nh
            bi = p % nbuf
            copies[bi].wait()
            if p == 0:
                vcopy.wait()
            buf = idxb.at[bi]

            nchain = 2

            def g_body(g, _, buf=buf, c=c, h=h):
                off = pl.multiple_of(
                    g * (nchain * _LANES), nchain * _LANES
                )
                zero = jnp.zeros((_LANES,), jnp.float32)

                def s_body(t, accs):
                    accs = list(accs)
                    for u in range(unroll):
                        s = t * unroll + u
                        for k in range(nchain):
                            iv = buf[s, pl.ds(off + k * _LANES, _LANES)]
                            accs[k] = accs[k] + plsc.load_gather(v_vmem, [iv])
                    return tuple(accs)

                trip = halves[h] // unroll
                if trip == 1:
                    accs = s_body(0, (zero,) * nchain)
                else:
                    accs = lax.fori_loop(0, trip, s_body, (zero,) * nchain)
                base = pl.multiple_of(
                    c * chunk_cols + off, nchain * _LANES
                )
                for k in range(nchain):
                    dk = pl.ds(base + k * _LANES, _LANES)
                    if h == 0:
                        out_vmem[dk] = accs[k]
                    elif h == nh - 1:
                        out_vmem[dk] = (out_vmem[dk] + accs[k]) * inv
                    else:
                        out_vmem[dk] = out_vmem[dk] + accs[k]
                return 0

            lax.fori_loop(0, groups // nchain, g_body, 0)
            if p + nbuf < n_pieces:
                copies[bi] = start(p + nbuf)
        pltpu.sync_copy(out_vmem, out_hbm.at[pl.ds(col0, cols_per_w)])

    return pool


def kernel(x, table, W, b):
    batch, seq = x.shape
    vocab, embed = table.shape
    v = _project(table.T, W, b, col_block=50176)
    pool = _make_pool(vocab, batch, seq, chunk_cols=128, unroll=40)
    return pool(v, x.T)


# speedup vs baseline: 1.1528x; 1.1528x over previous
import functools

import jax
import jax.numpy as jnp
from jax import lax
from jax.experimental import pallas as pl
from jax.experimental.pallas import tpu as pltpu
from jax.experimental.pallas import tpu_sc as plsc

_NC = 2
_NS = 16
_LANES = 16


def _proj_body(tableT_ref, w_ref, b_ref, v_ref):
    t = tableT_ref[...]
    w = w_ref[...]
    r = lax.dot_general(
        w, t, (((1,), (0,)), ((), ())), preferred_element_type=jnp.float32
    )
    v_ref[...] = jnp.reshape(r, (r.shape[1],)) + b_ref[0]


def _project(tableT, w_row, b, col_block):
    embed, vocab = tableT.shape
    grid = pl.cdiv(vocab, col_block)
    return pl.pallas_call(
        _proj_body,
        grid=(grid,),
        in_specs=[
            pl.BlockSpec((embed, col_block), lambda i: (0, i)),
            pl.BlockSpec((1, embed), lambda i: (0, 0)),
            pl.BlockSpec(memory_space=pltpu.SMEM),
        ],
        out_specs=pl.BlockSpec((col_block,), lambda i: (i,)),
        out_shape=jax.ShapeDtypeStruct((vocab,), jnp.float32),
    )(tableT, w_row, b)


def _make_pool(vocab, batch, seq, chunk_cols, unroll):
    nw = _NC * _NS
    cols_per_w = batch // nw
    n_chunks = cols_per_w // chunk_cols
    groups = chunk_cols // _LANES
    halves = (40, 40, 40, 40, 40)
    hoffs = (0, 40, 80, 120, 160)
    nh = len(halves)
    nbuf = 4
    n_pieces = nh * n_chunks
    mesh = plsc.VectorSubcoreMesh(core_axis_name="c", subcore_axis_name="s")

    @functools.partial(
        pl.kernel,
        out_type=jax.ShapeDtypeStruct((batch,), jnp.float32),
        mesh=mesh,
        compiler_params=pltpu.CompilerParams(needs_layout_passes=False),
        scratch_types=[
            pltpu.VMEM((vocab,), jnp.float32),
            pltpu.VMEM_SHARED((vocab,), jnp.float32),
            pltpu.VMEM((4, max(halves), chunk_cols), jnp.int32),
            pltpu.VMEM((cols_per_w,), jnp.float32),
            pltpu.SemaphoreType.DMA,
            pltpu.SemaphoreType.DMA,
            pltpu.SemaphoreType.DMA,
            pltpu.SemaphoreType.DMA,
            pltpu.SemaphoreType.DMA,
        ],
    )
    def pool(
        v_hbm, xT_hbm, out_hbm, v_vmem, v_shared, idxb, out_vmem,
        vsem, s0, s1, s2, s3,
    ):
        sid = lax.axis_index("s")
        wid = sid * _NC + lax.axis_index("c")
        col0 = wid * cols_per_w
        sems = (s0, s1, s2, s3)

        def start(p):
            c, h = p // nh, p % nh
            return pltpu.async_copy(
                xT_hbm.at[
                    pl.ds(hoffs[h], halves[h]),
                    pl.ds(col0 + c * chunk_cols, chunk_cols),
                ],
                idxb.at[p % nbuf, pl.ds(0, halves[h])],
                sems[p % nbuf],
            )

        inv = jnp.float32(1.0 / seq)
        copies = [None] * nbuf
        for p0 in range(nbuf):
            copies[p0] = start(p0)

        @pl.when(sid == 0)
        def _stage_v():
            pltpu.sync_copy(v_hbm, v_shared)

        plsc.subcore_barrier()
        vcopy = pltpu.async_copy(v_shared, v_vmem, vsem)
        for p in range(n_pieces):
            c, h = p // nh, p % nh
            bi = p % nbuf
            copies[bi].wait()
            if p == 0:
                vcopy.wait()
            buf = idxb.at[bi]

            nchain = 2

            def g_body(g, _, buf=buf, c=c, h=h):
                off = pl.multiple_of(
                    g * (nchain * _LANES), nchain * _LANES
                )
                zero = jnp.zeros((_LANES,), jnp.float32)

                def s_body(t, accs):
                    accs = list(accs)
                    for u in range(unroll):
                        s = t * unroll + u
                        for k in range(nchain):
                            iv = buf[s, pl.ds(off + k * _LANES, _LANES)]
                            accs[k] = accs[k] + plsc.load_gather(v_vmem, [iv])
                    return tuple(accs)

                trip = halves[h] // unroll
                if trip == 1:
                    accs = s_body(0, (zero,) * nchain)
                else:
                    accs = lax.fori_loop(0, trip, s_body, (zero,) * nchain)
                base = pl.multiple_of(
                    c * chunk_cols + off, nchain * _LANES
                )
                for k in range(nchain):
                    dk = pl.ds(base + k * _LANES, _LANES)
                    if h == 0:
                        out_vmem[dk] = accs[k]
                    elif h == nh - 1:
                        out_vmem[dk] = (out_vmem[dk] + accs[k]) * inv
                    else:
                        out_vmem[dk] = out_vmem[dk] + accs[k]
                return 0

            lax.fori_loop(0, groups // nchain, g_body, 0)
            if p + nbuf < n_pieces:
                copies[bi] = start(p + nbuf)
        pltpu.sync_copy(out_vmem, out_hbm.at[pl.ds(col0, cols_per_w)])

    return pool


def kernel(x, table, W, b):
    batch, seq = x.shape
    vocab, embed = table.shape
    v = _project(table.T, W, b, col_block=50176)
    pool = _make_pool(vocab, batch, seq, chunk_cols=128, unroll=8)
    return pool(v, x.T)
